# Initial kernel scaffold; baseline (speedup 1.0000x reference)
import jax
import jax.numpy as jnp
from jax.experimental import pallas as pl


def kernel(x, pos, batch, params):
    raise NotImplementedError("write your pallas kernel here")


# speedup vs baseline: 1.1671x; 1.1671x over previous
import jax
import jax.numpy as jnp
from jax.experimental import pallas as pl

N = 16384
K = 16
K_UP = 3
RATIO = 0.25
EPS = 1e-5


def _linear(p, x):
    y = x @ p["w"]
    if "b" in p:
        y = y + p["b"]
    return y


def _bnorm(x):
    m = jnp.mean(x, axis=0, keepdims=True)
    v = jnp.var(x, axis=0, keepdims=True)
    return (x - m) * jax.lax.rsqrt(v + EPS)


def _lin_norm_relu(p, x):
    return jax.nn.relu(_bnorm(_linear(p, x)))


def _mlp_list(ps, x):
    for p in ps:
        x = _lin_norm_relu(p, x)
    return x


def _knn_indices(db, q, k, chunk=4096):
    db2 = jnp.sum(db * db, axis=1)
    outs = []
    for i in range(0, q.shape[0], chunk):
        qc = q[i:i + chunk]
        d = jnp.sum(qc * qc, axis=1, keepdims=True) - 2.0 * (qc @ db.T) + db2[None, :]
        _, idx = jax.lax.top_k(-d, k)
        outs.append(idx)
    return jnp.concatenate(outs, axis=0)


def _strip_self(nbr17):
    n = nbr17.shape[0]
    mask = nbr17 == jnp.arange(n)[:, None]
    has = jnp.any(mask, axis=1)
    p_idx = jnp.where(has, jnp.argmax(mask, axis=1), nbr17.shape[1])
    j = jnp.arange(K)[None, :]
    take = j + (j >= p_idx[:, None]).astype(jnp.int32)
    return jnp.take_along_axis(nbr17, take, axis=1)


def _pt_conv(tp, x, pos, nbr):
    n = x.shape[0]
    srcs = jnp.concatenate([nbr, jnp.arange(n)[:, None]], axis=1)
    a_src = x @ tp["lin_src"]["w"]
    a_dst = x @ tp["lin_dst"]["w"]
    v = _linear(tp["lin"], x)
    rel = pos[:, None, :] - pos[srcs]
    c = x.shape[1]
    delta = _mlp_list(tp["pos_nn"], rel.reshape(-1, 3)).reshape(n, K + 1, c)
    u = a_dst[:, None, :] - a_src[srcs] + delta
    alpha = _mlp_list(tp["attn_nn"], u.reshape(n * (K + 1), c)).reshape(n, K + 1, c)
    m = jnp.max(alpha, axis=1, keepdims=True)
    e = jnp.exp(alpha - m)
    s = jnp.sum(e, axis=1, keepdims=True)
    alpha = e / (s + 1e-16)
    msg = alpha * (v[srcs] + delta)
    return jnp.max(msg, axis=1)


def _t_block(tp, x, pos, nbr):
    x = jax.nn.relu(_linear(tp["lin_in"], x))
    x = _pt_conv(tp, x, pos, nbr)
    return jax.nn.relu(_linear(tp["lin_out"], x))


def _fps(pos, n_samples):
    dists = jnp.sum((pos - pos[0]) ** 2, axis=1)
    idxs = jnp.zeros((n_samples,), jnp.int32)

    def body(i, state):
        dists, idxs = state
        nxt = jnp.argmax(dists).astype(jnp.int32)
        idxs = idxs.at[i].set(nxt)
        dists = jnp.minimum(dists, jnp.sum((pos - pos[nxt]) ** 2, axis=1))
        return (dists, idxs)

    _, idxs = jax.lax.fori_loop(1, n_samples, body, (dists, idxs))
    return idxs


def _knn_interpolate(x_sub, pos_sub, pos, k):
    idx = _knn_indices(pos_sub, pos, k)
    d = jnp.sum((pos[:, None, :] - pos_sub[idx]) ** 2, axis=-1)
    w = 1.0 / jnp.maximum(d, 1e-16)
    w = w / jnp.sum(w, axis=1, keepdims=True)
    return jnp.sum(w[..., None] * x_sub[idx], axis=1)


def kernel(x, pos, batch, params):
    p = params
    n = x.shape[0]
    x = _lin_norm_relu(p["mlp_input"], x)
    nbr17_0 = _knn_indices(pos, pos, K + 1)
    nbr0 = _strip_self(nbr17_0)
    x = _t_block(p["t_input"], x, pos, nbr0)
    x0, pos0 = x, pos

    idc = _fps(pos, int(n * RATIO))
    sub_pos = pos[idc]
    nbr_pool = nbr17_0[idc]
    h = _lin_norm_relu(p["down_mlp"][0], x)
    x = jnp.max(h[nbr_pool], axis=1)
    nbr17_1 = _knn_indices(sub_pos, sub_pos, K + 1)
    nbr1 = _strip_self(nbr17_1)
    x = _t_block(p["t_down"][0], x, sub_pos, nbr1)
    x1, pos1 = x, sub_pos

    x = jax.nn.relu(_linear(p["mlp_summit"], x))
    x = _t_block(p["t_summit"], x, pos1, nbr1)

    x_sub = _lin_norm_relu(p["up_mlp_sub"][0], x)
    xi = _knn_interpolate(x_sub, pos1, pos0, K_UP)
    x = _lin_norm_relu(p["up_mlp"][0], x0) + xi
    x = _t_block(p["t_up"][0], x, pos0, nbr0)

    out = jax.nn.relu(_linear(p["mlp_out1"], x))
    out = _linear(p["mlp_out2"], out)
    return jax.nn.log_softmax(out, axis=-1)


# speedup vs baseline: 3.4902x; 2.9905x over previous
import functools

import jax
import jax.numpy as jnp
from jax.experimental import pallas as pl
from jax.experimental.pallas import tpu as pltpu

N = 16384
K = 16
K_UP = 3
RATIO = 0.25
EPS = 1e-5


def _linear(p, x):
    y = x @ p["w"]
    if "b" in p:
        y = y + p["b"]
    return y


def _bnorm(x):
    m = jnp.mean(x, axis=0, keepdims=True)
    v = jnp.var(x, axis=0, keepdims=True)
    return (x - m) * jax.lax.rsqrt(v + EPS)


def _lin_norm_relu(p, x):
    return jax.nn.relu(_bnorm(_linear(p, x)))


def _mlp_list(ps, x):
    for p in ps:
        x = _lin_norm_relu(p, x)
    return x


def _knn_kernel(q_ref, dbt_ref, db2_ref, out_ref, d_scr, *, k, n_db, w):
    bq = q_ref.shape[0]
    nc = n_db // w
    qx = q_ref[:, 0:1]
    qy = q_ref[:, 1:2]
    qz = q_ref[:, 2:3]
    q2 = (qx * qx + qy * qy) + qz * qz
    qblk = q_ref[...]
    for c in range(nc):
        sl = pl.ds(c * w, w)
        qd = jax.lax.dot_general(
            qblk, dbt_ref[:, sl], (((1,), (0,)), ((), ())),
            precision=jax.lax.Precision.DEFAULT,
            preferred_element_type=jnp.float32)
        d_scr[:, sl] = q2 - 2.0 * qd + db2_ref[0:1, sl]

    iota_w = jax.lax.broadcasted_iota(jnp.int32, (bq, w), 1)
    lane = jax.lax.broadcasted_iota(jnp.int32, (bq, 128), 1)
    inf = jnp.float32(jnp.inf)

    def body(t, carry):
        lv, li, acc = carry
        m = jnp.full((bq, 1), inf, jnp.float32)
        am = jnp.full((bq, 1), n_db, jnp.int32)
        for c in range(nc):
            tile = d_scr[:, pl.ds(c * w, w)]
            gidx = iota_w + (c * w)
            ok = (tile > lv) | ((tile == lv) & (gidx > li))
            cand = jnp.where(ok, tile, inf)
            m_c = jnp.min(cand, axis=1, keepdims=True)
            am_c = jnp.min(jnp.where(cand == m_c, gidx, n_db),
                           axis=1, keepdims=True)
            better = m_c < m
            m = jnp.where(better, m_c, m)
            am = jnp.where(better, am_c, am)
        acc = jnp.where(lane == t, am, acc)
        return m, am, acc

    lv0 = jnp.full((bq, 1), -inf, jnp.float32)
    li0 = jnp.full((bq, 1), -1, jnp.int32)
    acc0 = jnp.zeros((bq, 128), jnp.int32)
    _, _, acc = jax.lax.fori_loop(0, k, body, (lv0, li0, acc0))
    out_ref[...] = acc


def _knn_topk(db, q, k, bq=256):
    n_db = db.shape[0]
    nq = q.shape[0]
    dbt = db.T
    db2 = jnp.sum(db * db, axis=1)[None, :]
    w = min(2048, n_db)
    out = pl.pallas_call(
        functools.partial(_knn_kernel, k=k, n_db=n_db, w=w),
        grid=(nq // bq,),
        in_specs=[
            pl.BlockSpec((bq, 3), lambda i: (i, 0)),
            pl.BlockSpec((3, n_db), lambda i: (0, 0)),
            pl.BlockSpec((1, n_db), lambda i: (0, 0)),
        ],
        out_specs=pl.BlockSpec((bq, 128), lambda i: (i, 0)),
        out_shape=jax.ShapeDtypeStruct((nq, 128), jnp.int32),
        scratch_shapes=[pltpu.VMEM((bq, n_db), jnp.float32)],
    )(q, dbt, db2)
    return out[:, :k]


def _fps_kernel(pt_ref, out_ref, *, n, n_samples):
    px = pt_ref[0:1, :]
    py = pt_ref[1:2, :]
    pz = pt_ref[2:3, :]
    iota = jax.lax.broadcasted_iota(jnp.int32, (1, n), 1)
    samp_iota = jax.lax.broadcasted_iota(jnp.int32, (1, n_samples), 1)
    dx = px - px[0, 0]
    dy = py - py[0, 0]
    dz = pz - pz[0, 0]
    d0 = (dx * dx + dy * dy) + dz * dz

    def body(i, carry):
        d, idxs = carry
        m = jnp.max(d)
        nxt = jnp.min(jnp.where(d == m, iota, n))
        xn = jnp.sum(jnp.where(iota == nxt, px, 0.0))
        yn = jnp.sum(jnp.where(iota == nxt, py, 0.0))
        zn = jnp.sum(jnp.where(iota == nxt, pz, 0.0))
        ex = px - xn
        ey = py - yn
        ez = pz - zn
        nd = (ex * ex + ey * ey) + ez * ez
        d = jnp.minimum(d, nd)
        idxs = jnp.where(samp_iota == i, nxt, idxs)
        return d, idxs

    _, idxs = jax.lax.fori_loop(
        1, n_samples, body, (d0, jnp.zeros((1, n_samples), jnp.int32)))
    out_ref[...] = idxs


def _fps(pos, n_samples):
    n = pos.shape[0]
    out = pl.pallas_call(
        functools.partial(_fps_kernel, n=n, n_samples=n_samples),
        out_shape=jax.ShapeDtypeStruct((1, n_samples), jnp.int32),
    )(pos.T)
    return out[0]


def _strip_self(nbr17):
    n = nbr17.shape[0]
    mask = nbr17 == jnp.arange(n)[:, None]
    has = jnp.any(mask, axis=1)
    p_idx = jnp.where(has, jnp.argmax(mask, axis=1), nbr17.shape[1])
    j = jnp.arange(K)[None, :]
    take = j + (j >= p_idx[:, None]).astype(jnp.int32)
    return jnp.take_along_axis(nbr17, take, axis=1)


def _pt_conv(tp, x, pos, nbr):
    n = x.shape[0]
    srcs = jnp.concatenate([nbr, jnp.arange(n)[:, None]], axis=1)
    a_src = x @ tp["lin_src"]["w"]
    a_dst = x @ tp["lin_dst"]["w"]
    v = _linear(tp["lin"], x)
    rel = pos[:, None, :] - pos[srcs]
    c = x.shape[1]
    delta = _mlp_list(tp["pos_nn"], rel.reshape(-1, 3)).reshape(n, K + 1, c)
    u = a_dst[:, None, :] - a_src[srcs] + delta
    alpha = _mlp_list(tp["attn_nn"], u.reshape(n * (K + 1), c)).reshape(n, K + 1, c)
    m = jnp.max(alpha, axis=1, keepdims=True)
    e = jnp.exp(alpha - m)
    s = jnp.sum(e, axis=1, keepdims=True)
    alpha = e / (s + 1e-16)
    msg = alpha * (v[srcs] + delta)
    return jnp.max(msg, axis=1)


def _t_block(tp, x, pos, nbr):
    x = jax.nn.relu(_linear(tp["lin_in"], x))
    x = _pt_conv(tp, x, pos, nbr)
    return jax.nn.relu(_linear(tp["lin_out"], x))


def _knn_interpolate(x_sub, pos_sub, pos, k):
    idx = _knn_topk(pos_sub, pos, k)
    d = jnp.sum((pos[:, None, :] - pos_sub[idx]) ** 2, axis=-1)
    w = 1.0 / jnp.maximum(d, 1e-16)
    w = w / jnp.sum(w, axis=1, keepdims=True)
    return jnp.sum(w[..., None] * x_sub[idx], axis=1)


def kernel(x, pos, batch, params):
    p = params
    n = x.shape[0]
    x = _lin_norm_relu(p["mlp_input"], x)
    nbr17_0 = _knn_topk(pos, pos, K + 1)
    nbr0 = _strip_self(nbr17_0)
    x = _t_block(p["t_input"], x, pos, nbr0)
    x0, pos0 = x, pos

    idc = _fps(pos, int(n * RATIO))
    sub_pos = pos[idc]
    nbr_pool = nbr17_0[idc]
    h = _lin_norm_relu(p["down_mlp"][0], x)
    x = jnp.max(h[nbr_pool], axis=1)
    nbr17_1 = _knn_topk(sub_pos, sub_pos, K + 1)
    nbr1 = _strip_self(nbr17_1)
    x = _t_block(p["t_down"][0], x, sub_pos, nbr1)

    pos1 = sub_pos
    x = jax.nn.relu(_linear(p["mlp_summit"], x))
    x = _t_block(p["t_summit"], x, pos1, nbr1)

    x_sub = _lin_norm_relu(p["up_mlp_sub"][0], x)
    xi = _knn_interpolate(x_sub, pos1, pos0, K_UP)
    x = _lin_norm_relu(p["up_mlp"][0], x0) + xi
    x = _t_block(p["t_up"][0], x, pos0, nbr0)

    out = jax.nn.relu(_linear(p["mlp_out1"], x))
    out = _linear(p["mlp_out2"], out)
    return jax.nn.log_softmax(out, axis=-1)


# speedup vs baseline: 4.2551x; 1.2191x over previous
import functools

import jax
import jax.numpy as jnp
from jax.experimental import pallas as pl
from jax.experimental.pallas import tpu as pltpu

N = 16384
K = 16
K_UP = 3
RATIO = 0.25
EPS = 1e-5


def _linear(p, x):
    y = x @ p["w"]
    if "b" in p:
        y = y + p["b"]
    return y


def _bnorm(x):
    m = jnp.mean(x, axis=0, keepdims=True)
    v = jnp.var(x, axis=0, keepdims=True)
    return (x - m) * jax.lax.rsqrt(v + EPS)


def _lin_norm_relu(p, x):
    return jax.nn.relu(_bnorm(_linear(p, x)))


def _mlp_list(ps, x):
    for p in ps:
        x = _lin_norm_relu(p, x)
    return x


def _knn_kernel(q_ref, dbt_ref, db2_ref, out_ref, d_scr, *, k, n_db, w):
    bq = q_ref.shape[0]
    nc = n_db // w
    qx = q_ref[:, 0:1]
    qy = q_ref[:, 1:2]
    qz = q_ref[:, 2:3]
    q2 = (qx * qx + qy * qy) + qz * qz
    qblk = q_ref[...]
    for c in range(nc):
        sl = pl.ds(c * w, w)
        qd = jax.lax.dot_general(
            qblk, dbt_ref[:, sl], (((1,), (0,)), ((), ())),
            precision=jax.lax.Precision.DEFAULT,
            preferred_element_type=jnp.float32)
        d_scr[:, sl] = q2 - 2.0 * qd + db2_ref[0:1, sl]

    iota_w = jax.lax.broadcasted_iota(jnp.int32, (bq, w), 1)
    lane = jax.lax.broadcasted_iota(jnp.int32, (bq, 128), 1)
    inf = jnp.float32(jnp.inf)

    def body(t, carry):
        lv, li, acc = carry
        m = jnp.full((bq, 1), inf, jnp.float32)
        am = jnp.full((bq, 1), n_db, jnp.int32)
        for c in range(nc):
            tile = d_scr[:, pl.ds(c * w, w)]
            gidx = iota_w + (c * w)
            ok = (tile > lv) | ((tile == lv) & (gidx > li))
            cand = jnp.where(ok, tile, inf)
            m_c = jnp.min(cand, axis=1, keepdims=True)
            am_c = jnp.min(jnp.where(cand == m_c, gidx, n_db),
                           axis=1, keepdims=True)
            better = m_c < m
            m = jnp.where(better, m_c, m)
            am = jnp.where(better, am_c, am)
        acc = jnp.where(lane == t, am, acc)
        return m, am, acc

    lv0 = jnp.full((bq, 1), -inf, jnp.float32)
    li0 = jnp.full((bq, 1), -1, jnp.int32)
    acc0 = jnp.zeros((bq, 128), jnp.int32)
    _, _, acc = jax.lax.fori_loop(0, k, body, (lv0, li0, acc0))
    out_ref[...] = acc


def _knn_topk(db, q, k, bq=256):
    n_db = db.shape[0]
    nq = q.shape[0]
    dbt = db.T
    db2 = jnp.sum(db * db, axis=1)[None, :]
    w = min(2048, n_db)
    out = pl.pallas_call(
        functools.partial(_knn_kernel, k=k, n_db=n_db, w=w),
        grid=(nq // bq,),
        in_specs=[
            pl.BlockSpec((bq, 3), lambda i: (i, 0)),
            pl.BlockSpec((3, n_db), lambda i: (0, 0)),
            pl.BlockSpec((1, n_db), lambda i: (0, 0)),
        ],
        out_specs=pl.BlockSpec((bq, 128), lambda i: (i, 0)),
        out_shape=jax.ShapeDtypeStruct((nq, 128), jnp.int32),
        scratch_shapes=[pltpu.VMEM((bq, n_db), jnp.float32)],
    )(q, dbt, db2)
    return out[:, :k]


def _fps_kernel(pt_ref, out_ref, *, n, n_samples):
    px = pt_ref[0:1, :]
    py = pt_ref[1:2, :]
    pz = pt_ref[2:3, :]
    iota = jax.lax.broadcasted_iota(jnp.int32, (1, n), 1)
    samp_iota = jax.lax.broadcasted_iota(jnp.int32, (1, n_samples), 1)
    dx = px - px[0, 0]
    dy = py - py[0, 0]
    dz = pz - pz[0, 0]
    d0 = (dx * dx + dy * dy) + dz * dz

    def body(i, carry):
        d, idxs = carry
        m = jnp.max(d)
        nxt = jnp.min(jnp.where(d == m, iota, n))
        xn = jnp.sum(jnp.where(iota == nxt, px, 0.0))
        yn = jnp.sum(jnp.where(iota == nxt, py, 0.0))
        zn = jnp.sum(jnp.where(iota == nxt, pz, 0.0))
        ex = px - xn
        ey = py - yn
        ez = pz - zn
        nd = (ex * ex + ey * ey) + ez * ez
        d = jnp.minimum(d, nd)
        idxs = jnp.where(samp_iota == i, nxt, idxs)
        return d, idxs

    _, idxs = jax.lax.fori_loop(
        1, n_samples, body, (d0, jnp.zeros((1, n_samples), jnp.int32)))
    out_ref[...] = idxs


def _fps(pos, n_samples):
    n = pos.shape[0]
    out = pl.pallas_call(
        functools.partial(_fps_kernel, n=n, n_samples=n_samples),
        out_shape=jax.ShapeDtypeStruct((1, n_samples), jnp.int32),
    )(pos.T)
    return out[0]


KP = K + 1


def _pt_chain(upto, c, b, refs, consts):
    e = KP * b
    dot = functools.partial(
        jax.lax.dot_general,
        dimension_numbers=(((1,), (0,)), ((), ())),
        preferred_element_type=jnp.float32)
    y1 = dot(refs["rel"][...].reshape(e, 3), consts["W1"][...]) + consts["b1"][...]
    if upto == 1:
        return y1
    mu1 = consts["n1"][0:1, :]
    r1 = consts["n1"][1:2, :]
    h1 = jax.nn.relu((y1 - mu1) * r1)
    y2 = dot(h1, consts["W2"][...]) + consts["b2"][...]
    if upto == 2:
        return y2
    mu2 = consts["n2"][0:1, :c]
    r2 = consts["n2"][1:2, :c]
    delta = jax.nn.relu((y2 - mu2) * r2)
    delta3 = delta.reshape(KP, b, c)
    adiff3 = refs["adst"][...][None, :, :] - refs["asrc"][...]
    u = (adiff3 + delta3).reshape(e, c)
    z1 = dot(u, consts["A1"][...]) + consts["a1"][...]
    if upto == 3:
        return z1
    mu3 = consts["n3"][0:1, :]
    r3 = consts["n3"][1:2, :]
    g1 = jax.nn.relu((z1 - mu3) * r3)
    z2 = dot(g1, consts["A2"][...]) + consts["a2"][...]
    if upto == 4:
        return z2
    mu4 = consts["n4"][0:1, :c]
    r4 = consts["n4"][1:2, :c]
    araw3 = jax.nn.relu((z2 - mu4) * r4).reshape(KP, b, c)
    m = jnp.max(araw3, axis=0)
    ex = jnp.exp(araw3 - m[None, :, :])
    s = jnp.sum(ex, axis=0)
    alpha = ex / (s + 1e-16)[None, :, :]
    msg = alpha * (refs["vg"][...] + delta3)
    return jnp.max(msg, axis=0)


def _pt_stage_kernel(*args, stage, c, b, in_names, const_names):
    nin = len(in_names) + len(const_names)
    in_refs = dict(zip(in_names, args[:len(in_names)]))
    consts = dict(zip(const_names, args[len(in_names):nin]))
    out_ref = args[nin]
    if stage == 5:
        out_ref[...] = _pt_chain(5, c, b, in_refs, consts)
        return
    y = _pt_chain(stage, c, b, in_refs, consts)
    part = jnp.concatenate(
        [jnp.sum(y, axis=0, keepdims=True),
         jnp.sum(y * y, axis=0, keepdims=True)], axis=0)

    @pl.when(pl.program_id(0) == 0)
    def _():
        out_ref[...] = jnp.zeros_like(out_ref)

    out_ref[...] += part


def _pt_conv_pallas(tp, x, pos, nbr):
    n, c = x.shape
    b = 256
    srcs_t = jnp.concatenate([nbr, jnp.arange(n)[:, None]], axis=1).T
    a_src = x @ tp["lin_src"]["w"]
    a_dst = x @ tp["lin_dst"]["w"]
    v = _linear(tp["lin"], x)
    rel = pos[None, :, :] - pos[srcs_t]
    asrc_g = a_src[srcs_t]
    v_g = v[srcs_t]

    W1 = tp["pos_nn"][0]["w"]
    b1 = tp["pos_nn"][0]["b"][None, :]
    W2 = tp["pos_nn"][1]["w"]
    b2 = tp["pos_nn"][1]["b"][None, :]
    A1 = tp["attn_nn"][0]["w"]
    a1 = tp["attn_nn"][0]["b"][None, :]
    A2 = tp["attn_nn"][1]["w"]
    a2 = tp["attn_nn"][1]["b"][None, :]
    h = W1.shape[1]
    e_tot = KP * n

    edge_spec = lambda width: pl.BlockSpec((KP, b, width), lambda i: (0, i, 0))
    node_spec = lambda width: pl.BlockSpec((b, width), lambda i: (i, 0))
    full = lambda a: pl.BlockSpec(a.shape, lambda i: (0, 0))

    def norm_of(stat, width):
        s = stat[0, :width]
        ss = stat[1, :width]
        mu = s / e_tot
        var = ss / e_tot - mu * mu
        return jnp.stack([mu, jax.lax.rsqrt(var + EPS)])

    ins_by_stage = {
        1: ["rel"],
        2: ["rel"],
        3: ["rel", "asrc", "adst"],
        4: ["rel", "asrc", "adst"],
        5: ["rel", "asrc", "adst", "vg"],
    }
    arrays = {"rel": rel, "asrc": asrc_g, "adst": a_dst, "vg": v_g}
    specs = {"rel": edge_spec(3), "asrc": edge_spec(c),
             "adst": node_spec(c), "vg": edge_spec(c)}
    consts_avail = {"W1": W1, "b1": b1, "W2": W2, "b2": b2,
                    "A1": A1, "a1": a1, "A2": A2, "a2": a2}
    const_by_stage = {
        1: ["W1", "b1"],
        2: ["W1", "b1", "W2", "b2", "n1"],
        3: ["W1", "b1", "W2", "b2", "A1", "a1", "n1", "n2"],
        4: ["W1", "b1", "W2", "b2", "A1", "a1", "A2", "a2", "n1", "n2", "n3"],
        5: ["W1", "b1", "W2", "b2", "A1", "a1", "A2", "a2",
            "n1", "n2", "n3", "n4"],
    }
    stat_width = {1: h, 2: c, 3: h, 4: c}

    norms = {}
    for stage in (1, 2, 3, 4, 5):
        in_names = ins_by_stage[stage]
        const_names = const_by_stage[stage]
        cvals = []
        for nm in const_names:
            if nm.startswith("n") and nm[1:].isdigit():
                cvals.append(norms[nm])
            else:
                cvals.append(consts_avail[nm])
        if stage == 5:
            out_spec = pl.BlockSpec((b, c), lambda i: (i, 0))
            out_shape = jax.ShapeDtypeStruct((n, c), jnp.float32)
        else:
            w = stat_width[stage]
            out_spec = pl.BlockSpec((2, w), lambda i: (0, 0))
            out_shape = jax.ShapeDtypeStruct((2, w), jnp.float32)
        res = pl.pallas_call(
            functools.partial(_pt_stage_kernel, stage=stage, c=c, b=b,
                              in_names=in_names, const_names=const_names),
            grid=(n // b,),
            in_specs=[specs[nm] for nm in in_names] + [full(cv) for cv in cvals],
            out_specs=out_spec,
            out_shape=out_shape,
        )(*[arrays[nm] for nm in in_names], *cvals)
        if stage == 5:
            return res
        width = c if stage in (2, 4) else h
        norms[f"n{stage}"] = norm_of(res, width)


def _strip_self(nbr17):
    n = nbr17.shape[0]
    mask = nbr17 == jnp.arange(n)[:, None]
    has = jnp.any(mask, axis=1)
    p_idx = jnp.where(has, jnp.argmax(mask, axis=1), nbr17.shape[1])
    j = jnp.arange(K)[None, :]
    take = j + (j >= p_idx[:, None]).astype(jnp.int32)
    return jnp.take_along_axis(nbr17, take, axis=1)


def _pt_conv(tp, x, pos, nbr):
    n = x.shape[0]
    srcs = jnp.concatenate([nbr, jnp.arange(n)[:, None]], axis=1)
    a_src = x @ tp["lin_src"]["w"]
    a_dst = x @ tp["lin_dst"]["w"]
    v = _linear(tp["lin"], x)
    rel = pos[:, None, :] - pos[srcs]
    c = x.shape[1]
    delta = _mlp_list(tp["pos_nn"], rel.reshape(-1, 3)).reshape(n, K + 1, c)
    u = a_dst[:, None, :] - a_src[srcs] + delta
    alpha = _mlp_list(tp["attn_nn"], u.reshape(n * (K + 1), c)).reshape(n, K + 1, c)
    m = jnp.max(alpha, axis=1, keepdims=True)
    e = jnp.exp(alpha - m)
    s = jnp.sum(e, axis=1, keepdims=True)
    alpha = e / (s + 1e-16)
    msg = alpha * (v[srcs] + delta)
    return jnp.max(msg, axis=1)


def _t_block(tp, x, pos, nbr):
    x = jax.nn.relu(_linear(tp["lin_in"], x))
    x = _pt_conv_pallas(tp, x, pos, nbr)
    return jax.nn.relu(_linear(tp["lin_out"], x))


def _knn_interpolate(x_sub, pos_sub, pos, k):
    idx = _knn_topk(pos_sub, pos, k)
    d = jnp.sum((pos[:, None, :] - pos_sub[idx]) ** 2, axis=-1)
    w = 1.0 / jnp.maximum(d, 1e-16)
    w = w / jnp.sum(w, axis=1, keepdims=True)
    return jnp.sum(w[..., None] * x_sub[idx], axis=1)


def kernel(x, pos, batch, params):
    p = params
    n = x.shape[0]
    x = _lin_norm_relu(p["mlp_input"], x)
    nbr17_0 = _knn_topk(pos, pos, K + 1)
    nbr0 = _strip_self(nbr17_0)
    x = _t_block(p["t_input"], x, pos, nbr0)
    x0, pos0 = x, pos

    idc = _fps(pos, int(n * RATIO))
    sub_pos = pos[idc]
    nbr_pool = nbr17_0[idc]
    h = _lin_norm_relu(p["down_mlp"][0], x)
    x = jnp.max(h[nbr_pool], axis=1)
    nbr17_1 = _knn_topk(sub_pos, sub_pos, K + 1)
    nbr1 = _strip_self(nbr17_1)
    x = _t_block(p["t_down"][0], x, sub_pos, nbr1)

    pos1 = sub_pos
    x = jax.nn.relu(_linear(p["mlp_summit"], x))
    x = _t_block(p["t_summit"], x, pos1, nbr1)

    x_sub = _lin_norm_relu(p["up_mlp_sub"][0], x)
    xi = _knn_interpolate(x_sub, pos1, pos0, K_UP)
    x = _lin_norm_relu(p["up_mlp"][0], x0) + xi
    x = _t_block(p["t_up"][0], x, pos0, nbr0)

    out = jax.nn.relu(_linear(p["mlp_out1"], x))
    out = _linear(p["mlp_out2"], out)
    return jax.nn.log_softmax(out, axis=-1)


# speedup vs baseline: 4.4670x; 1.0498x over previous
import functools

import jax
import jax.numpy as jnp
from jax.experimental import pallas as pl
from jax.experimental.pallas import tpu as pltpu

N = 16384
K = 16
K_UP = 3
RATIO = 0.25
EPS = 1e-5


def _linear(p, x):
    y = x @ p["w"]
    if "b" in p:
        y = y + p["b"]
    return y


def _bnorm(x):
    m = jnp.mean(x, axis=0, keepdims=True)
    v = jnp.var(x, axis=0, keepdims=True)
    return (x - m) * jax.lax.rsqrt(v + EPS)


def _lin_norm_relu(p, x):
    return jax.nn.relu(_bnorm(_linear(p, x)))


def _mlp_list(ps, x):
    for p in ps:
        x = _lin_norm_relu(p, x)
    return x


def _knn_kernel(q_ref, dbt_ref, db2_ref, out_ref, d_scr, *, k, n_db, w):
    bq = q_ref.shape[0]
    nc = n_db // w
    qx = q_ref[:, 0:1]
    qy = q_ref[:, 1:2]
    qz = q_ref[:, 2:3]
    q2 = (qx * qx + qy * qy) + qz * qz
    qblk = q_ref[...]
    for c in range(nc):
        sl = pl.ds(c * w, w)
        qd = jax.lax.dot_general(
            qblk, dbt_ref[:, sl], (((1,), (0,)), ((), ())),
            precision=jax.lax.Precision.DEFAULT,
            preferred_element_type=jnp.float32)
        d_scr[:, sl] = q2 - 2.0 * qd + db2_ref[0:1, sl]

    iota_w = jax.lax.broadcasted_iota(jnp.int32, (bq, w), 1)
    lane = jax.lax.broadcasted_iota(jnp.int32, (bq, 128), 1)
    inf = jnp.float32(jnp.inf)

    def body(t, carry):
        lv, li, acc = carry
        m = jnp.full((bq, 1), inf, jnp.float32)
        am = jnp.full((bq, 1), n_db, jnp.int32)
        for c in range(nc):
            tile = d_scr[:, pl.ds(c * w, w)]
            gidx = iota_w + (c * w)
            ok = (tile > lv) | ((tile == lv) & (gidx > li))
            cand = jnp.where(ok, tile, inf)
            m_c = jnp.min(cand, axis=1, keepdims=True)
            am_c = jnp.min(jnp.where(cand == m_c, gidx, n_db),
                           axis=1, keepdims=True)
            better = m_c < m
            m = jnp.where(better, m_c, m)
            am = jnp.where(better, am_c, am)
        acc = jnp.where(lane == t, am, acc)
        return m, am, acc

    lv0 = jnp.full((bq, 1), -inf, jnp.float32)
    li0 = jnp.full((bq, 1), -1, jnp.int32)
    acc0 = jnp.zeros((bq, 128), jnp.int32)
    _, _, acc = jax.lax.fori_loop(0, k, body, (lv0, li0, acc0))
    out_ref[...] = acc


def _knn_topk(db, q, k, bq=256):
    n_db = db.shape[0]
    nq = q.shape[0]
    dbt = db.T
    db2 = jnp.sum(db * db, axis=1)[None, :]
    w = min(2048, n_db)
    out = pl.pallas_call(
        functools.partial(_knn_kernel, k=k, n_db=n_db, w=w),
        grid=(nq // bq,),
        in_specs=[
            pl.BlockSpec((bq, 3), lambda i: (i, 0)),
            pl.BlockSpec((3, n_db), lambda i: (0, 0)),
            pl.BlockSpec((1, n_db), lambda i: (0, 0)),
        ],
        out_specs=pl.BlockSpec((bq, 128), lambda i: (i, 0)),
        out_shape=jax.ShapeDtypeStruct((nq, 128), jnp.int32),
        scratch_shapes=[pltpu.VMEM((bq, n_db), jnp.float32)],
    )(q, dbt, db2)
    return out[:, :k]


def _fps_kernel(pt_ref, pn_ref, out_ref, *, n, n_samples):
    px = pt_ref[0:1, :]
    py = pt_ref[1:2, :]
    pz = pt_ref[2:3, :]
    iota = jax.lax.broadcasted_iota(jnp.int32, (1, n), 1)
    samp_iota = jax.lax.broadcasted_iota(jnp.int32, (1, n_samples), 1)
    dx = px - px[0, 0]
    dy = py - py[0, 0]
    dz = pz - pz[0, 0]
    d0 = (dx * dx + dy * dy) + dz * dz

    def body(i, carry):
        d, idxs = carry
        m = jnp.max(d)
        nxt = jnp.min(jnp.where(d == m, iota, n))
        row = pn_ref[pl.ds(nxt, 1), :]
        ex = px - row[0, 0]
        ey = py - row[0, 1]
        ez = pz - row[0, 2]
        nd = (ex * ex + ey * ey) + ez * ez
        d = jnp.minimum(d, nd)
        idxs = jnp.where(samp_iota == i, nxt, idxs)
        return d, idxs

    _, idxs = jax.lax.fori_loop(
        1, n_samples, body, (d0, jnp.zeros((1, n_samples), jnp.int32)))
    out_ref[...] = idxs


def _fps(pos, n_samples):
    n = pos.shape[0]
    out = pl.pallas_call(
        functools.partial(_fps_kernel, n=n, n_samples=n_samples),
        out_shape=jax.ShapeDtypeStruct((1, n_samples), jnp.int32),
    )(pos.T, pos)
    return out[0]


KP = K + 1


def _pt_chain_slab(upto, j, c, refs, consts):
    dotT = functools.partial(
        jax.lax.dot_general,
        dimension_numbers=(((0,), (0,)), ((), ())),
        preferred_element_type=jnp.float32)
    y1 = dotT(consts["W1"][...], refs["rel"][j]) + consts["b1"][...]
    if upto == 1:
        return y1
    mu1 = consts["n1"][:, 0:1]
    r1 = consts["n1"][:, 1:2]
    h1 = jax.nn.relu((y1 - mu1) * r1)
    y2 = dotT(consts["W2"][...], h1) + consts["b2"][...]
    if upto == 2:
        return y2
    mu2 = consts["n2"][:, 0:1]
    r2 = consts["n2"][:, 1:2]
    delta = jax.nn.relu((y2 - mu2) * r2)
    u = (refs["adst"][...] - refs["asrc"][j]) + delta
    z1 = dotT(consts["A1"][...], u) + consts["a1"][...]
    if upto == 3:
        return z1
    mu3 = consts["n3"][:, 0:1]
    r3 = consts["n3"][:, 1:2]
    g1 = jax.nn.relu((z1 - mu3) * r3)
    z2 = dotT(consts["A2"][...], g1) + consts["a2"][...]
    if upto == 4:
        return z2
    mu4 = consts["n4"][:, 0:1]
    r4 = consts["n4"][:, 1:2]
    araw = jax.nn.relu((z2 - mu4) * r4)
    return araw, delta


def _pt_stage_kernel(*args, stage, c, b, in_names, const_names):
    nin = len(in_names) + len(const_names)
    in_refs = dict(zip(in_names, args[:len(in_names)]))
    consts = dict(zip(const_names, args[len(in_names):nin]))
    out_ref = args[nin]
    if stage == 5:
        araws, deltas = [], []
        for j in range(KP):
            a_j, d_j = _pt_chain_slab(5, j, c, in_refs, consts)
            araws.append(a_j)
            deltas.append(d_j)
        m = araws[0]
        for j in range(1, KP):
            m = jnp.maximum(m, araws[j])
        exs = [jnp.exp(a_j - m) for a_j in araws]
        s = exs[0]
        for j in range(1, KP):
            s = s + exs[j]
        sden = s + 1e-16
        out = None
        for j in range(KP):
            msg = (exs[j] / sden) * (in_refs["vg"][j] + deltas[j])
            out = msg if out is None else jnp.maximum(out, msg)
        out_ref[...] = out
        return
    ssum = None
    ssq = None
    for j in range(KP):
        y = _pt_chain_slab(stage, j, c, in_refs, consts)
        ps = jnp.sum(y, axis=1, keepdims=True)
        pq = jnp.sum(y * y, axis=1, keepdims=True)
        ssum = ps if ssum is None else ssum + ps
        ssq = pq if ssq is None else ssq + pq
    part = jnp.concatenate([ssum, ssq], axis=1)

    @pl.when(pl.program_id(0) == 0)
    def _():
        out_ref[...] = jnp.zeros_like(out_ref)

    out_ref[...] += part


def _pt_conv_pallas(tp, x, pos, nbr):
    n, c = x.shape
    b = 512
    srcs_t = jnp.concatenate([nbr, jnp.arange(n)[:, None]], axis=1).T
    a_src = x @ tp["lin_src"]["w"]
    a_dst = x @ tp["lin_dst"]["w"]
    v = _linear(tp["lin"], x)
    rel = jnp.transpose(pos[None, :, :] - pos[srcs_t], (0, 2, 1))
    asrc_g = jnp.transpose(a_src[srcs_t], (0, 2, 1))
    v_g = jnp.transpose(v[srcs_t], (0, 2, 1))
    adst_t = a_dst.T

    W1 = tp["pos_nn"][0]["w"]
    b1 = tp["pos_nn"][0]["b"][:, None]
    W2 = tp["pos_nn"][1]["w"]
    b2 = tp["pos_nn"][1]["b"][:, None]
    A1 = tp["attn_nn"][0]["w"]
    a1 = tp["attn_nn"][0]["b"][:, None]
    A2 = tp["attn_nn"][1]["w"]
    a2 = tp["attn_nn"][1]["b"][:, None]
    h = W1.shape[1]
    e_tot = KP * n

    edge_spec = lambda width: pl.BlockSpec((KP, width, b), lambda i: (0, 0, i))
    node_spec = lambda width: pl.BlockSpec((width, b), lambda i: (0, i))

    def full(a):
        nd = len(a.shape)
        if nd == 2:
            return pl.BlockSpec(a.shape, lambda i: (0, 0))
        return pl.BlockSpec(a.shape, lambda i: (0,) * nd)

    def norm_of(stat):
        s = stat[:, 0]
        ss = stat[:, 1]
        mu = s / e_tot
        var = ss / e_tot - mu * mu
        return jnp.stack([mu, jax.lax.rsqrt(var + EPS)], axis=1)

    ins_by_stage = {
        1: ["rel"],
        2: ["rel"],
        3: ["rel", "asrc", "adst"],
        4: ["rel", "asrc", "adst"],
        5: ["rel", "asrc", "adst", "vg"],
    }
    arrays = {"rel": rel, "asrc": asrc_g, "adst": adst_t, "vg": v_g}
    specs = {"rel": edge_spec(3), "asrc": edge_spec(c),
             "adst": node_spec(c), "vg": edge_spec(c)}
    consts_avail = {"W1": W1, "b1": b1, "W2": W2, "b2": b2,
                    "A1": A1, "a1": a1, "A2": A2, "a2": a2}
    const_by_stage = {
        1: ["W1", "b1"],
        2: ["W1", "b1", "W2", "b2", "n1"],
        3: ["W1", "b1", "W2", "b2", "A1", "a1", "n1", "n2"],
        4: ["W1", "b1", "W2", "b2", "A1", "a1", "A2", "a2", "n1", "n2", "n3"],
        5: ["W1", "b1", "W2", "b2", "A1", "a1", "A2", "a2",
            "n1", "n2", "n3", "n4"],
    }
    stat_width = {1: h, 2: c, 3: h, 4: c}

    norms = {}
    for stage in (1, 2, 3, 4, 5):
        in_names = ins_by_stage[stage]
        const_names = const_by_stage[stage]
        cvals = []
        for nm in const_names:
            if nm.startswith("n") and nm[1:].isdigit():
                cvals.append(norms[nm])
            else:
                cvals.append(consts_avail[nm])
        if stage == 5:
            out_spec = pl.BlockSpec((c, b), lambda i: (0, i))
            out_shape = jax.ShapeDtypeStruct((c, n), jnp.float32)
        else:
            w = stat_width[stage]
            out_spec = pl.BlockSpec((w, 2), lambda i: (0, 0))
            out_shape = jax.ShapeDtypeStruct((w, 2), jnp.float32)
        res = pl.pallas_call(
            functools.partial(_pt_stage_kernel, stage=stage, c=c, b=b,
                              in_names=in_names, const_names=const_names),
            grid=(n // b,),
            in_specs=[specs[nm] for nm in in_names] + [full(cv) for cv in cvals],
            out_specs=out_spec,
            out_shape=out_shape,
        )(*[arrays[nm] for nm in in_names], *cvals)
        if stage == 5:
            return res.T
        norms[f"n{stage}"] = norm_of(res)


def _strip_self(nbr17):
    n = nbr17.shape[0]
    mask = nbr17 == jnp.arange(n)[:, None]
    has = jnp.any(mask, axis=1)
    p_idx = jnp.where(has, jnp.argmax(mask, axis=1), nbr17.shape[1])
    j = jnp.arange(K)[None, :]
    take = j + (j >= p_idx[:, None]).astype(jnp.int32)
    return jnp.take_along_axis(nbr17, take, axis=1)


def _pt_conv(tp, x, pos, nbr):
    n = x.shape[0]
    srcs = jnp.concatenate([nbr, jnp.arange(n)[:, None]], axis=1)
    a_src = x @ tp["lin_src"]["w"]
    a_dst = x @ tp["lin_dst"]["w"]
    v = _linear(tp["lin"], x)
    rel = pos[:, None, :] - pos[srcs]
    c = x.shape[1]
    delta = _mlp_list(tp["pos_nn"], rel.reshape(-1, 3)).reshape(n, K + 1, c)
    u = a_dst[:, None, :] - a_src[srcs] + delta
    alpha = _mlp_list(tp["attn_nn"], u.reshape(n * (K + 1), c)).reshape(n, K + 1, c)
    m = jnp.max(alpha, axis=1, keepdims=True)
    e = jnp.exp(alpha - m)
    s = jnp.sum(e, axis=1, keepdims=True)
    alpha = e / (s + 1e-16)
    msg = alpha * (v[srcs] + delta)
    return jnp.max(msg, axis=1)


def _t_block(tp, x, pos, nbr):
    x = jax.nn.relu(_linear(tp["lin_in"], x))
    x = _pt_conv_pallas(tp, x, pos, nbr)
    return jax.nn.relu(_linear(tp["lin_out"], x))


def _knn_interpolate(x_sub, pos_sub, pos, k):
    idx = _knn_topk(pos_sub, pos, k)
    d = jnp.sum((pos[:, None, :] - pos_sub[idx]) ** 2, axis=-1)
    w = 1.0 / jnp.maximum(d, 1e-16)
    w = w / jnp.sum(w, axis=1, keepdims=True)
    return jnp.sum(w[..., None] * x_sub[idx], axis=1)


def kernel(x, pos, batch, params):
    p = params
    n = x.shape[0]
    x = _lin_norm_relu(p["mlp_input"], x)
    nbr17_0 = _knn_topk(pos, pos, K + 1)
    nbr0 = _strip_self(nbr17_0)
    x = _t_block(p["t_input"], x, pos, nbr0)
    x0, pos0 = x, pos

    idc = _fps(pos, int(n * RATIO))
    sub_pos = pos[idc]
    nbr_pool = nbr17_0[idc]
    h = _lin_norm_relu(p["down_mlp"][0], x)
    x = jnp.max(h[nbr_pool], axis=1)
    nbr17_1 = _knn_topk(sub_pos, sub_pos, K + 1)
    nbr1 = _strip_self(nbr17_1)
    x = _t_block(p["t_down"][0], x, sub_pos, nbr1)

    pos1 = sub_pos
    x = jax.nn.relu(_linear(p["mlp_summit"], x))
    x = _t_block(p["t_summit"], x, pos1, nbr1)

    x_sub = _lin_norm_relu(p["up_mlp_sub"][0], x)
    xi = _knn_interpolate(x_sub, pos1, pos0, K_UP)
    x = _lin_norm_relu(p["up_mlp"][0], x0) + xi
    x = _t_block(p["t_up"][0], x, pos0, nbr0)

    out = jax.nn.relu(_linear(p["mlp_out1"], x))
    out = _linear(p["mlp_out2"], out)
    return jax.nn.log_softmax(out, axis=-1)


# speedup vs baseline: 4.6486x; 1.0406x over previous
import functools

import jax
import jax.numpy as jnp
from jax.experimental import pallas as pl
from jax.experimental.pallas import tpu as pltpu

N = 16384
K = 16
K_UP = 3
RATIO = 0.25
EPS = 1e-5


def _linear(p, x):
    y = x @ p["w"]
    if "b" in p:
        y = y + p["b"]
    return y


def _bnorm(x):
    m = jnp.mean(x, axis=0, keepdims=True)
    v = jnp.var(x, axis=0, keepdims=True)
    return (x - m) * jax.lax.rsqrt(v + EPS)


def _lin_norm_relu(p, x):
    return jax.nn.relu(_bnorm(_linear(p, x)))


def _mlp_list(ps, x):
    for p in ps:
        x = _lin_norm_relu(p, x)
    return x


def _knn_kernel(q_ref, dbt_ref, db2_ref, out_ref, d_scr, *, k, n_db, w):
    bq = q_ref.shape[0]
    nc = n_db // w
    qx = q_ref[:, 0:1]
    qy = q_ref[:, 1:2]
    qz = q_ref[:, 2:3]
    q2 = (qx * qx + qy * qy) + qz * qz
    qblk = q_ref[...]
    for c in range(nc):
        sl = pl.ds(c * w, w)
        qd = jax.lax.dot_general(
            qblk, dbt_ref[:, sl], (((1,), (0,)), ((), ())),
            precision=jax.lax.Precision.DEFAULT,
            preferred_element_type=jnp.float32)
        d_scr[:, sl] = q2 - 2.0 * qd + db2_ref[0:1, sl]

    iota_w = jax.lax.broadcasted_iota(jnp.int32, (bq, w), 1)
    lane = jax.lax.broadcasted_iota(jnp.int32, (bq, 128), 1)
    inf = jnp.float32(jnp.inf)

    def body(t, carry):
        lv, li, acc = carry
        m = jnp.full((bq, 1), inf, jnp.float32)
        am = jnp.full((bq, 1), n_db, jnp.int32)
        for c in range(nc):
            tile = d_scr[:, pl.ds(c * w, w)]
            gidx = iota_w + (c * w)
            ok = (tile > lv) | ((tile == lv) & (gidx > li))
            cand = jnp.where(ok, tile, inf)
            m_c = jnp.min(cand, axis=1, keepdims=True)
            am_c = jnp.min(jnp.where(cand == m_c, gidx, n_db),
                           axis=1, keepdims=True)
            better = m_c < m
            m = jnp.where(better, m_c, m)
            am = jnp.where(better, am_c, am)
        acc = jnp.where(lane == t, am, acc)
        return m, am, acc

    lv0 = jnp.full((bq, 1), -inf, jnp.float32)
    li0 = jnp.full((bq, 1), -1, jnp.int32)
    acc0 = jnp.zeros((bq, 128), jnp.int32)
    _, _, acc = jax.lax.fori_loop(0, k, body, (lv0, li0, acc0))
    out_ref[...] = acc


def _knn_topk(db, q, k, bq=512):
    n_db = db.shape[0]
    nq = q.shape[0]
    dbt = db.T
    db2 = jnp.sum(db * db, axis=1)[None, :]
    w = min(2048, n_db)
    out = pl.pallas_call(
        functools.partial(_knn_kernel, k=k, n_db=n_db, w=w),
        grid=(nq // bq,),
        in_specs=[
            pl.BlockSpec((bq, 3), lambda i: (i, 0)),
            pl.BlockSpec((3, n_db), lambda i: (0, 0)),
            pl.BlockSpec((1, n_db), lambda i: (0, 0)),
        ],
        out_specs=pl.BlockSpec((bq, 128), lambda i: (i, 0)),
        out_shape=jax.ShapeDtypeStruct((nq, 128), jnp.int32),
        scratch_shapes=[pltpu.VMEM((bq, n_db), jnp.float32)],
    )(q, dbt, db2)
    return out[:, :k]


def _fps_kernel(pt_ref, pn_ref, out_ref, *, n, n_samples):
    px = pt_ref[0:1, :]
    py = pt_ref[1:2, :]
    pz = pt_ref[2:3, :]
    iota = jax.lax.broadcasted_iota(jnp.int32, (1, n), 1)
    samp_iota = jax.lax.broadcasted_iota(jnp.int32, (1, n_samples), 1)
    dx = px - px[0, 0]
    dy = py - py[0, 0]
    dz = pz - pz[0, 0]
    d0 = (dx * dx + dy * dy) + dz * dz

    def body(i, carry):
        d, idxs = carry
        m = jnp.max(d)
        nxt = jnp.min(jnp.where(d == m, iota, n))
        row = pn_ref[pl.ds(nxt, 1), :]
        ex = px - row[0, 0]
        ey = py - row[0, 1]
        ez = pz - row[0, 2]
        nd = (ex * ex + ey * ey) + ez * ez
        d = jnp.minimum(d, nd)
        idxs = jnp.where(samp_iota == i, nxt, idxs)
        return d, idxs

    _, idxs = jax.lax.fori_loop(
        1, n_samples, body, (d0, jnp.zeros((1, n_samples), jnp.int32)))
    out_ref[...] = idxs


def _fps(pos, n_samples):
    n = pos.shape[0]
    out = pl.pallas_call(
        functools.partial(_fps_kernel, n=n, n_samples=n_samples),
        out_shape=jax.ShapeDtypeStruct((1, n_samples), jnp.int32),
    )(pos.T, pos)
    return out[0]


KP = K + 1


def _pt_chain_slab(upto, j, c, refs, consts):
    dotT = functools.partial(
        jax.lax.dot_general,
        dimension_numbers=(((0,), (0,)), ((), ())),
        preferred_element_type=jnp.float32)
    y1 = dotT(consts["W1"][...], refs["rel"][j]) + consts["b1"][...]
    if upto == 1:
        return y1
    mu1 = consts["n1"][:, 0:1]
    r1 = consts["n1"][:, 1:2]
    h1 = jax.nn.relu((y1 - mu1) * r1)
    y2 = dotT(consts["W2"][...], h1) + consts["b2"][...]
    if upto == 2:
        return y2
    mu2 = consts["n2"][:, 0:1]
    r2 = consts["n2"][:, 1:2]
    delta = jax.nn.relu((y2 - mu2) * r2)
    u = (refs["adst"][...] - refs["asrc"][j]) + delta
    z1 = dotT(consts["A1"][...], u) + consts["a1"][...]
    if upto == 3:
        return z1
    mu3 = consts["n3"][:, 0:1]
    r3 = consts["n3"][:, 1:2]
    g1 = jax.nn.relu((z1 - mu3) * r3)
    z2 = dotT(consts["A2"][...], g1) + consts["a2"][...]
    if upto == 4:
        return z2
    mu4 = consts["n4"][:, 0:1]
    r4 = consts["n4"][:, 1:2]
    araw = jax.nn.relu((z2 - mu4) * r4)
    return araw, delta


def _pt_stage_kernel(*args, stage, c, b, in_names, const_names):
    nin = len(in_names) + len(const_names)
    in_refs = dict(zip(in_names, args[:len(in_names)]))
    consts = dict(zip(const_names, args[len(in_names):nin]))
    out_ref = args[nin]
    if stage == 5:
        araws, deltas = [], []
        for j in range(KP):
            a_j, d_j = _pt_chain_slab(5, j, c, in_refs, consts)
            araws.append(a_j)
            deltas.append(d_j)
        m = araws[0]
        for j in range(1, KP):
            m = jnp.maximum(m, araws[j])
        exs = [jnp.exp(a_j - m) for a_j in araws]
        s = exs[0]
        for j in range(1, KP):
            s = s + exs[j]
        sden = s + 1e-16
        out = None
        for j in range(KP):
            msg = (exs[j] / sden) * (in_refs["vg"][j] + deltas[j])
            out = msg if out is None else jnp.maximum(out, msg)
        out_ref[...] = out
        return
    ssum = None
    ssq = None
    for j in range(KP):
        y = _pt_chain_slab(stage, j, c, in_refs, consts)
        ps = jnp.sum(y, axis=1, keepdims=True)
        pq = jnp.sum(y * y, axis=1, keepdims=True)
        ssum = ps if ssum is None else ssum + ps
        ssq = pq if ssq is None else ssq + pq
    part = jnp.concatenate([ssum, ssq], axis=1)

    @pl.when(pl.program_id(0) == 0)
    def _():
        out_ref[...] = jnp.zeros_like(out_ref)

    out_ref[...] += part


def _pt_conv_pallas(tp, x, pos, nbr):
    n, c = x.shape
    b = 1024
    srcs_t = jnp.concatenate([nbr, jnp.arange(n)[:, None]], axis=1).T
    a_src = x @ tp["lin_src"]["w"]
    a_dst = x @ tp["lin_dst"]["w"]
    v = _linear(tp["lin"], x)
    rel = jnp.transpose(pos[None, :, :] - pos[srcs_t], (0, 2, 1))
    asrc_g = jnp.transpose(a_src[srcs_t], (0, 2, 1))
    v_g = jnp.transpose(v[srcs_t], (0, 2, 1))
    adst_t = a_dst.T

    W1 = tp["pos_nn"][0]["w"]
    b1 = tp["pos_nn"][0]["b"][:, None]
    W2 = tp["pos_nn"][1]["w"]
    b2 = tp["pos_nn"][1]["b"][:, None]
    A1 = tp["attn_nn"][0]["w"]
    a1 = tp["attn_nn"][0]["b"][:, None]
    A2 = tp["attn_nn"][1]["w"]
    a2 = tp["attn_nn"][1]["b"][:, None]
    h = W1.shape[1]
    e_tot = KP * n

    edge_spec = lambda width: pl.BlockSpec((KP, width, b), lambda i: (0, 0, i))
    node_spec = lambda width: pl.BlockSpec((width, b), lambda i: (0, i))

    def full(a):
        nd = len(a.shape)
        if nd == 2:
            return pl.BlockSpec(a.shape, lambda i: (0, 0))
        return pl.BlockSpec(a.shape, lambda i: (0,) * nd)

    def norm_of(stat):
        s = stat[:, 0]
        ss = stat[:, 1]
        mu = s / e_tot
        var = ss / e_tot - mu * mu
        return jnp.stack([mu, jax.lax.rsqrt(var + EPS)], axis=1)

    ins_by_stage = {
        1: ["rel"],
        2: ["rel"],
        3: ["rel", "asrc", "adst"],
        4: ["rel", "asrc", "adst"],
        5: ["rel", "asrc", "adst", "vg"],
    }
    arrays = {"rel": rel, "asrc": asrc_g, "adst": adst_t, "vg": v_g}
    specs = {"rel": edge_spec(3), "asrc": edge_spec(c),
             "adst": node_spec(c), "vg": edge_spec(c)}
    consts_avail = {"W1": W1, "b1": b1, "W2": W2, "b2": b2,
                    "A1": A1, "a1": a1, "A2": A2, "a2": a2}
    const_by_stage = {
        1: ["W1", "b1"],
        2: ["W1", "b1", "W2", "b2", "n1"],
        3: ["W1", "b1", "W2", "b2", "A1", "a1", "n1", "n2"],
        4: ["W1", "b1", "W2", "b2", "A1", "a1", "A2", "a2", "n1", "n2", "n3"],
        5: ["W1", "b1", "W2", "b2", "A1", "a1", "A2", "a2",
            "n1", "n2", "n3", "n4"],
    }
    stat_width = {1: h, 2: c, 3: h, 4: c}

    norms = {}
    for stage in (1, 2, 3, 4, 5):
        in_names = ins_by_stage[stage]
        const_names = const_by_stage[stage]
        cvals = []
        for nm in const_names:
            if nm.startswith("n") and nm[1:].isdigit():
                cvals.append(norms[nm])
            else:
                cvals.append(consts_avail[nm])
        if stage == 5:
            out_spec = pl.BlockSpec((c, b), lambda i: (0, i))
            out_shape = jax.ShapeDtypeStruct((c, n), jnp.float32)
        else:
            w = stat_width[stage]
            out_spec = pl.BlockSpec((w, 2), lambda i: (0, 0))
            out_shape = jax.ShapeDtypeStruct((w, 2), jnp.float32)
        res = pl.pallas_call(
            functools.partial(_pt_stage_kernel, stage=stage, c=c, b=b,
                              in_names=in_names, const_names=const_names),
            grid=(n // b,),
            in_specs=[specs[nm] for nm in in_names] + [full(cv) for cv in cvals],
            out_specs=out_spec,
            out_shape=out_shape,
        )(*[arrays[nm] for nm in in_names], *cvals)
        if stage == 5:
            return res.T
        norms[f"n{stage}"] = norm_of(res)


def _strip_self(nbr17):
    n = nbr17.shape[0]
    mask = nbr17 == jnp.arange(n)[:, None]
    has = jnp.any(mask, axis=1)
    p_idx = jnp.where(has, jnp.argmax(mask, axis=1), nbr17.shape[1])
    j = jnp.arange(K)[None, :]
    take = j + (j >= p_idx[:, None]).astype(jnp.int32)
    return jnp.take_along_axis(nbr17, take, axis=1)


def _pt_conv(tp, x, pos, nbr):
    n = x.shape[0]
    srcs = jnp.concatenate([nbr, jnp.arange(n)[:, None]], axis=1)
    a_src = x @ tp["lin_src"]["w"]
    a_dst = x @ tp["lin_dst"]["w"]
    v = _linear(tp["lin"], x)
    rel = pos[:, None, :] - pos[srcs]
    c = x.shape[1]
    delta = _mlp_list(tp["pos_nn"], rel.reshape(-1, 3)).reshape(n, K + 1, c)
    u = a_dst[:, None, :] - a_src[srcs] + delta
    alpha = _mlp_list(tp["attn_nn"], u.reshape(n * (K + 1), c)).reshape(n, K + 1, c)
    m = jnp.max(alpha, axis=1, keepdims=True)
    e = jnp.exp(alpha - m)
    s = jnp.sum(e, axis=1, keepdims=True)
    alpha = e / (s + 1e-16)
    msg = alpha * (v[srcs] + delta)
    return jnp.max(msg, axis=1)


def _t_block(tp, x, pos, nbr):
    x = jax.nn.relu(_linear(tp["lin_in"], x))
    x = _pt_conv_pallas(tp, x, pos, nbr)
    return jax.nn.relu(_linear(tp["lin_out"], x))


def _knn_interpolate(x_sub, pos_sub, pos, k):
    idx = _knn_topk(pos_sub, pos, k)
    d = jnp.sum((pos[:, None, :] - pos_sub[idx]) ** 2, axis=-1)
    w = 1.0 / jnp.maximum(d, 1e-16)
    w = w / jnp.sum(w, axis=1, keepdims=True)
    return jnp.sum(w[..., None] * x_sub[idx], axis=1)


def kernel(x, pos, batch, params):
    p = params
    n = x.shape[0]
    x = _lin_norm_relu(p["mlp_input"], x)
    nbr17_0 = _knn_topk(pos, pos, K + 1)
    nbr0 = _strip_self(nbr17_0)
    x = _t_block(p["t_input"], x, pos, nbr0)
    x0, pos0 = x, pos

    idc = _fps(pos, int(n * RATIO))
    sub_pos = pos[idc]
    nbr_pool = nbr17_0[idc]
    h = _lin_norm_relu(p["down_mlp"][0], x)
    x = jnp.max(h[nbr_pool], axis=1)
    nbr17_1 = _knn_topk(sub_pos, sub_pos, K + 1)
    nbr1 = _strip_self(nbr17_1)
    x = _t_block(p["t_down"][0], x, sub_pos, nbr1)

    pos1 = sub_pos
    x = jax.nn.relu(_linear(p["mlp_summit"], x))
    x = _t_block(p["t_summit"], x, pos1, nbr1)

    x_sub = _lin_norm_relu(p["up_mlp_sub"][0], x)
    xi = _knn_interpolate(x_sub, pos1, pos0, K_UP)
    x = _lin_norm_relu(p["up_mlp"][0], x0) + xi
    x = _t_block(p["t_up"][0], x, pos0, nbr0)

    out = jax.nn.relu(_linear(p["mlp_out1"], x))
    out = _linear(p["mlp_out2"], out)
    return jax.nn.log_softmax(out, axis=-1)


# speedup vs baseline: 4.7697x; 1.0261x over previous
import functools

import jax
import jax.numpy as jnp
from jax.experimental import pallas as pl
from jax.experimental.pallas import tpu as pltpu

N = 16384
K = 16
K_UP = 3
RATIO = 0.25
EPS = 1e-5


def _linear(p, x):
    y = x @ p["w"]
    if "b" in p:
        y = y + p["b"]
    return y


def _bnorm(x):
    m = jnp.mean(x, axis=0, keepdims=True)
    v = jnp.var(x, axis=0, keepdims=True)
    return (x - m) * jax.lax.rsqrt(v + EPS)


def _lin_norm_relu(p, x):
    return jax.nn.relu(_bnorm(_linear(p, x)))


def _mlp_list(ps, x):
    for p in ps:
        x = _lin_norm_relu(p, x)
    return x


def _knn_kernel(q_ref, dbt_ref, db2_ref, out_ref, d_scr, *, k, n_db, w):
    bq = q_ref.shape[0]
    nc = n_db // w
    qx = q_ref[:, 0:1]
    qy = q_ref[:, 1:2]
    qz = q_ref[:, 2:3]
    q2 = (qx * qx + qy * qy) + qz * qz
    qblk = q_ref[...]
    for c in range(nc):
        sl = pl.ds(c * w, w)
        qd = jax.lax.dot_general(
            qblk, dbt_ref[:, sl], (((1,), (0,)), ((), ())),
            precision=jax.lax.Precision.DEFAULT,
            preferred_element_type=jnp.float32)
        d_scr[:, sl] = q2 - 2.0 * qd + db2_ref[0:1, sl]

    iota_w = jax.lax.broadcasted_iota(jnp.int32, (bq, w), 1)
    lane = jax.lax.broadcasted_iota(jnp.int32, (bq, 128), 1)
    inf = jnp.float32(jnp.inf)

    def body(t, carry):
        lv, li, acc = carry
        m = jnp.full((bq, 1), inf, jnp.float32)
        am = jnp.full((bq, 1), n_db, jnp.int32)
        for c in range(nc):
            tile = d_scr[:, pl.ds(c * w, w)]
            gidx = iota_w + (c * w)
            ok = (tile > lv) | ((tile == lv) & (gidx > li))
            cand = jnp.where(ok, tile, inf)
            m_c = jnp.min(cand, axis=1, keepdims=True)
            am_c = jnp.min(jnp.where(cand == m_c, gidx, n_db),
                           axis=1, keepdims=True)
            better = m_c < m
            m = jnp.where(better, m_c, m)
            am = jnp.where(better, am_c, am)
        acc = jnp.where(lane == t, am, acc)
        return m, am, acc

    lv0 = jnp.full((bq, 1), -inf, jnp.float32)
    li0 = jnp.full((bq, 1), -1, jnp.int32)
    acc0 = jnp.zeros((bq, 128), jnp.int32)
    _, _, acc = jax.lax.fori_loop(0, k, body, (lv0, li0, acc0))
    out_ref[...] = acc


def _knn_topk(db, q, k, bq=512):
    n_db = db.shape[0]
    nq = q.shape[0]
    dbt = db.T
    db2 = jnp.sum(db * db, axis=1)[None, :]
    w = min(2048, n_db)
    out = pl.pallas_call(
        functools.partial(_knn_kernel, k=k, n_db=n_db, w=w),
        grid=(nq // bq,),
        in_specs=[
            pl.BlockSpec((bq, 3), lambda i: (i, 0)),
            pl.BlockSpec((3, n_db), lambda i: (0, 0)),
            pl.BlockSpec((1, n_db), lambda i: (0, 0)),
        ],
        out_specs=pl.BlockSpec((bq, 128), lambda i: (i, 0)),
        out_shape=jax.ShapeDtypeStruct((nq, 128), jnp.int32),
        scratch_shapes=[pltpu.VMEM((bq, n_db), jnp.float32)],
    )(q, dbt, db2)
    return out[:, :k]


def _fps_kernel(pr_ref, pn_ref, out_ref, *, n, n_samples, rows):
    cols = n // rows
    px = pr_ref[0]
    py = pr_ref[1]
    pz = pr_ref[2]
    col_iota = jax.lax.broadcasted_iota(jnp.int32, (rows, cols), 1)
    row_iota = jax.lax.broadcasted_iota(jnp.int32, (rows, cols), 0)
    iota = row_iota * cols + col_iota
    samp_iota = jax.lax.broadcasted_iota(jnp.int32, (1, n_samples), 1)
    dx = px - pn_ref[0, 0]
    dy = py - pn_ref[0, 1]
    dz = pz - pn_ref[0, 2]
    d0 = (dx * dx + dy * dy) + dz * dz

    def body(i, carry):
        d, idxs = carry
        m = jnp.max(d)
        nxt = jnp.min(jnp.where(d == m, iota, n))
        row = pn_ref[pl.ds(nxt, 1), :]
        ex = px - row[0, 0]
        ey = py - row[0, 1]
        ez = pz - row[0, 2]
        nd = (ex * ex + ey * ey) + ez * ez
        d = jnp.minimum(d, nd)
        idxs = jnp.where(samp_iota == i, nxt, idxs)
        return d, idxs

    _, idxs = jax.lax.fori_loop(
        1, n_samples, body, (d0, jnp.zeros((1, n_samples), jnp.int32)))
    out_ref[...] = idxs


def _fps(pos, n_samples, rows=8):
    n = pos.shape[0]
    pos_r = pos.T.reshape(3, rows, n // rows)
    out = pl.pallas_call(
        functools.partial(_fps_kernel, n=n, n_samples=n_samples, rows=rows),
        out_shape=jax.ShapeDtypeStruct((1, n_samples), jnp.int32),
    )(pos_r, pos)
    return out[0]


KP = K + 1


def _pt_chain_slab(upto, j, c, refs, consts):
    dotT = functools.partial(
        jax.lax.dot_general,
        dimension_numbers=(((0,), (0,)), ((), ())),
        preferred_element_type=jnp.float32)
    y1 = dotT(consts["W1"][...], refs["rel"][j]) + consts["b1"][...]
    if upto == 1:
        return y1
    mu1 = consts["n1"][:, 0:1]
    r1 = consts["n1"][:, 1:2]
    h1 = jax.nn.relu((y1 - mu1) * r1)
    y2 = dotT(consts["W2"][...], h1) + consts["b2"][...]
    if upto == 2:
        return y2
    mu2 = consts["n2"][:, 0:1]
    r2 = consts["n2"][:, 1:2]
    delta = jax.nn.relu((y2 - mu2) * r2)
    u = (refs["adst"][...] - refs["asrc"][j]) + delta
    z1 = dotT(consts["A1"][...], u) + consts["a1"][...]
    if upto == 3:
        return z1
    mu3 = consts["n3"][:, 0:1]
    r3 = consts["n3"][:, 1:2]
    g1 = jax.nn.relu((z1 - mu3) * r3)
    z2 = dotT(consts["A2"][...], g1) + consts["a2"][...]
    if upto == 4:
        return z2
    mu4 = consts["n4"][:, 0:1]
    r4 = consts["n4"][:, 1:2]
    araw = jax.nn.relu((z2 - mu4) * r4)
    return araw, delta


def _pt_stage_kernel(*args, stage, c, b, in_names, const_names):
    nin = len(in_names) + len(const_names)
    in_refs = dict(zip(in_names, args[:len(in_names)]))
    consts = dict(zip(const_names, args[len(in_names):nin]))
    out_ref = args[nin]
    if stage == 5:
        araws, deltas = [], []
        for j in range(KP):
            a_j, d_j = _pt_chain_slab(5, j, c, in_refs, consts)
            araws.append(a_j)
            deltas.append(d_j)
        m = araws[0]
        for j in range(1, KP):
            m = jnp.maximum(m, araws[j])
        exs = [jnp.exp(a_j - m) for a_j in araws]
        s = exs[0]
        for j in range(1, KP):
            s = s + exs[j]
        sden = s + 1e-16
        out = None
        for j in range(KP):
            msg = (exs[j] / sden) * (in_refs["vg"][j] + deltas[j])
            out = msg if out is None else jnp.maximum(out, msg)
        out_ref[...] = out
        return
    ssum = None
    ssq = None
    for j in range(KP):
        y = _pt_chain_slab(stage, j, c, in_refs, consts)
        ps = jnp.sum(y, axis=1, keepdims=True)
        pq = jnp.sum(y * y, axis=1, keepdims=True)
        ssum = ps if ssum is None else ssum + ps
        ssq = pq if ssq is None else ssq + pq
    part = jnp.concatenate([ssum, ssq], axis=1)

    @pl.when(pl.program_id(0) == 0)
    def _():
        out_ref[...] = jnp.zeros_like(out_ref)

    out_ref[...] += part


def _pt_conv_pallas(tp, x, pos, nbr):
    n, c = x.shape
    b = 1024
    srcs_t = jnp.concatenate([nbr, jnp.arange(n)[:, None]], axis=1).T
    a_src = x @ tp["lin_src"]["w"]
    a_dst = x @ tp["lin_dst"]["w"]
    v = _linear(tp["lin"], x)
    rel = jnp.transpose(pos[None, :, :] - pos[srcs_t], (0, 2, 1))
    asrc_g = jnp.transpose(a_src[srcs_t], (0, 2, 1))
    v_g = jnp.transpose(v[srcs_t], (0, 2, 1))
    adst_t = a_dst.T

    W1 = tp["pos_nn"][0]["w"]
    b1 = tp["pos_nn"][0]["b"][:, None]
    W2 = tp["pos_nn"][1]["w"]
    b2 = tp["pos_nn"][1]["b"][:, None]
    A1 = tp["attn_nn"][0]["w"]
    a1 = tp["attn_nn"][0]["b"][:, None]
    A2 = tp["attn_nn"][1]["w"]
    a2 = tp["attn_nn"][1]["b"][:, None]
    h = W1.shape[1]
    e_tot = KP * n

    edge_spec = lambda width: pl.BlockSpec((KP, width, b), lambda i: (0, 0, i))
    node_spec = lambda width: pl.BlockSpec((width, b), lambda i: (0, i))

    def full(a):
        nd = len(a.shape)
        if nd == 2:
            return pl.BlockSpec(a.shape, lambda i: (0, 0))
        return pl.BlockSpec(a.shape, lambda i: (0,) * nd)

    def norm_of(stat):
        s = stat[:, 0]
        ss = stat[:, 1]
        mu = s / e_tot
        var = ss / e_tot - mu * mu
        return jnp.stack([mu, jax.lax.rsqrt(var + EPS)], axis=1)

    ins_by_stage = {
        1: ["rel"],
        2: ["rel"],
        3: ["rel", "asrc", "adst"],
        4: ["rel", "asrc", "adst"],
        5: ["rel", "asrc", "adst", "vg"],
    }
    arrays = {"rel": rel, "asrc": asrc_g, "adst": adst_t, "vg": v_g}
    specs = {"rel": edge_spec(3), "asrc": edge_spec(c),
             "adst": node_spec(c), "vg": edge_spec(c)}
    consts_avail = {"W1": W1, "b1": b1, "W2": W2, "b2": b2,
                    "A1": A1, "a1": a1, "A2": A2, "a2": a2}
    const_by_stage = {
        1: ["W1", "b1"],
        2: ["W1", "b1", "W2", "b2", "n1"],
        3: ["W1", "b1", "W2", "b2", "A1", "a1", "n1", "n2"],
        4: ["W1", "b1", "W2", "b2", "A1", "a1", "A2", "a2", "n1", "n2", "n3"],
        5: ["W1", "b1", "W2", "b2", "A1", "a1", "A2", "a2",
            "n1", "n2", "n3", "n4"],
    }
    stat_width = {1: h, 2: c, 3: h, 4: c}

    norms = {}
    for stage in (1, 2, 3, 4, 5):
        in_names = ins_by_stage[stage]
        const_names = const_by_stage[stage]
        cvals = []
        for nm in const_names:
            if nm.startswith("n") and nm[1:].isdigit():
                cvals.append(norms[nm])
            else:
                cvals.append(consts_avail[nm])
        if stage == 5:
            out_spec = pl.BlockSpec((c, b), lambda i: (0, i))
            out_shape = jax.ShapeDtypeStruct((c, n), jnp.float32)
        else:
            w = stat_width[stage]
            out_spec = pl.BlockSpec((w, 2), lambda i: (0, 0))
            out_shape = jax.ShapeDtypeStruct((w, 2), jnp.float32)
        res = pl.pallas_call(
            functools.partial(_pt_stage_kernel, stage=stage, c=c, b=b,
                              in_names=in_names, const_names=const_names),
            grid=(n // b,),
            in_specs=[specs[nm] for nm in in_names] + [full(cv) for cv in cvals],
            out_specs=out_spec,
            out_shape=out_shape,
        )(*[arrays[nm] for nm in in_names], *cvals)
        if stage == 5:
            return res.T
        norms[f"n{stage}"] = norm_of(res)


def _strip_self(nbr17):
    n = nbr17.shape[0]
    mask = nbr17 == jnp.arange(n)[:, None]
    has = jnp.any(mask, axis=1)
    p_idx = jnp.where(has, jnp.argmax(mask, axis=1), nbr17.shape[1])
    j = jnp.arange(K)[None, :]
    take = j + (j >= p_idx[:, None]).astype(jnp.int32)
    return jnp.take_along_axis(nbr17, take, axis=1)


def _pt_conv(tp, x, pos, nbr):
    n = x.shape[0]
    srcs = jnp.concatenate([nbr, jnp.arange(n)[:, None]], axis=1)
    a_src = x @ tp["lin_src"]["w"]
    a_dst = x @ tp["lin_dst"]["w"]
    v = _linear(tp["lin"], x)
    rel = pos[:, None, :] - pos[srcs]
    c = x.shape[1]
    delta = _mlp_list(tp["pos_nn"], rel.reshape(-1, 3)).reshape(n, K + 1, c)
    u = a_dst[:, None, :] - a_src[srcs] + delta
    alpha = _mlp_list(tp["attn_nn"], u.reshape(n * (K + 1), c)).reshape(n, K + 1, c)
    m = jnp.max(alpha, axis=1, keepdims=True)
    e = jnp.exp(alpha - m)
    s = jnp.sum(e, axis=1, keepdims=True)
    alpha = e / (s + 1e-16)
    msg = alpha * (v[srcs] + delta)
    return jnp.max(msg, axis=1)


def _t_block(tp, x, pos, nbr):
    x = jax.nn.relu(_linear(tp["lin_in"], x))
    x = _pt_conv_pallas(tp, x, pos, nbr)
    return jax.nn.relu(_linear(tp["lin_out"], x))


def _knn_interpolate(x_sub, pos_sub, pos, k):
    idx = _knn_topk(pos_sub, pos, k)
    d = jnp.sum((pos[:, None, :] - pos_sub[idx]) ** 2, axis=-1)
    w = 1.0 / jnp.maximum(d, 1e-16)
    w = w / jnp.sum(w, axis=1, keepdims=True)
    return jnp.sum(w[..., None] * x_sub[idx], axis=1)


def kernel(x, pos, batch, params):
    p = params
    n = x.shape[0]
    x = _lin_norm_relu(p["mlp_input"], x)
    nbr17_0 = _knn_topk(pos, pos, K + 1)
    nbr0 = _strip_self(nbr17_0)
    x = _t_block(p["t_input"], x, pos, nbr0)
    x0, pos0 = x, pos

    idc = _fps(pos, int(n * RATIO))
    sub_pos = pos[idc]
    nbr_pool = nbr17_0[idc]
    h = _lin_norm_relu(p["down_mlp"][0], x)
    x = jnp.max(h[nbr_pool], axis=1)
    nbr17_1 = _knn_topk(sub_pos, sub_pos, K + 1)
    nbr1 = _strip_self(nbr17_1)
    x = _t_block(p["t_down"][0], x, sub_pos, nbr1)

    pos1 = sub_pos
    x = jax.nn.relu(_linear(p["mlp_summit"], x))
    x = _t_block(p["t_summit"], x, pos1, nbr1)

    x_sub = _lin_norm_relu(p["up_mlp_sub"][0], x)
    xi = _knn_interpolate(x_sub, pos1, pos0, K_UP)
    x = _lin_norm_relu(p["up_mlp"][0], x0) + xi
    x = _t_block(p["t_up"][0], x, pos0, nbr0)

    out = jax.nn.relu(_linear(p["mlp_out1"], x))
    out = _linear(p["mlp_out2"], out)
    return jax.nn.log_softmax(out, axis=-1)
